# initial kernel scaffold (unmeasured)
import jax
import jax.numpy as jnp
from jax import lax
from jax.experimental import pallas as pl
from jax.experimental.pallas import tpu as pltpu

W = 32
M = 4096
N = 8192
CH = M // W


def _gemm(x, w_mat, scale_x, scale_w):
    m, k = x.shape
    _, n = w_mat.shape
    BM = 512

    def body(sx_ref, sw_ref, x_ref, w_ref, o_ref):
        s = sx_ref[0] * sw_ref[0]
        xb = x_ref[...].astype(jnp.bfloat16)
        wb = w_ref[...].astype(jnp.bfloat16)
        acc = lax.dot_general(
            xb, wb, (((1,), (0,)), ((), ())),
            preferred_element_type=jnp.float32,
        )
        o_ref[...] = acc * s

    return pl.pallas_call(
        body,
        grid=(m // BM,),
        in_specs=[
            pl.BlockSpec(memory_space=pltpu.SMEM),
            pl.BlockSpec(memory_space=pltpu.SMEM),
            pl.BlockSpec((BM, k), lambda i: (i, 0)),
            pl.BlockSpec((k, n), lambda i: (0, 0)),
        ],
        out_specs=pl.BlockSpec((BM, n), lambda i: (i, 0)),
        out_shape=jax.ShapeDtypeStruct((m, n), jnp.float32),
    )(scale_x, scale_w, x, w_mat)


def _allreduce_body(p_ref, o_ref, recv_ref, acc_ref, tmp_ref,
                    send_sems, recv_sems, dma_sems, cred_sem):
    my = lax.axis_index("i")
    left = lax.rem(my + W - 1, W)
    right = lax.rem(my + 1, W)

    init = pltpu.make_async_copy(p_ref, o_ref, dma_sems.at[0])
    init.start()
    init.wait()

    for step in range(2 * (W - 1)):
        slot = step % 2
        if step < W - 1:
            c_send = lax.rem(my - step + 2 * W, W)
            c_recv = lax.rem(my - step - 1 + 2 * W, W)
        else:
            t = step - (W - 1)
            c_send = lax.rem(my + 1 - t + 2 * W, W)
            c_recv = lax.rem(my - t + 2 * W, W)

        if step >= 2:
            pl.semaphore_wait(cred_sem, 1)

        rdma = pltpu.make_async_remote_copy(
            src_ref=o_ref.at[pl.ds(c_send * CH, CH)],
            dst_ref=recv_ref.at[slot],
            send_sem=send_sems.at[slot],
            recv_sem=recv_sems.at[slot],
            device_id=(right,),
            device_id_type=pl.DeviceIdType.MESH,
        )
        rdma.start()
        rdma.wait()

        if step < W - 1:
            cp_a = pltpu.make_async_copy(
                o_ref.at[pl.ds(c_recv * CH, CH)], acc_ref, dma_sems.at[0])
            cp_b = pltpu.make_async_copy(
                recv_ref.at[slot], tmp_ref, dma_sems.at[1])
            cp_a.start()
            cp_b.start()
            cp_a.wait()
            cp_b.wait()
            acc_ref[...] = acc_ref[...] + tmp_ref[...]
            cp_c = pltpu.make_async_copy(
                acc_ref, o_ref.at[pl.ds(c_recv * CH, CH)], dma_sems.at[0])
            cp_c.start()
            cp_c.wait()
        else:
            cp_c = pltpu.make_async_copy(
                recv_ref.at[slot], o_ref.at[pl.ds(c_recv * CH, CH)],
                dma_sems.at[0])
            cp_c.start()
            cp_c.wait()

        pl.semaphore_signal(
            cred_sem, inc=1,
            device_id=(left,), device_id_type=pl.DeviceIdType.MESH)

    pl.semaphore_wait(cred_sem, 2)


def kernel(x, w_mat, scale_x, scale_w):
    partial = _gemm(x, w_mat, scale_x, scale_w)
    out = pl.pallas_call(
        _allreduce_body,
        out_shape=jax.ShapeDtypeStruct((M, N), jnp.float32),
        in_specs=[pl.BlockSpec(memory_space=pltpu.ANY)],
        out_specs=pl.BlockSpec(memory_space=pltpu.ANY),
        scratch_shapes=[
            pltpu.ANY((2, CH, N), jnp.float32),
            pltpu.VMEM((CH, N), jnp.float32),
            pltpu.VMEM((CH, N), jnp.float32),
            pltpu.SemaphoreType.DMA((2,)),
            pltpu.SemaphoreType.DMA((2,)),
            pltpu.SemaphoreType.DMA((2,)),
            pltpu.SemaphoreType.REGULAR,
        ],
        compiler_params=pltpu.CompilerParams(collective_id=0),
    )(partial)
    return out


# baseline (device time: 7347316 ns/iter reference)
import jax
import jax.numpy as jnp
from jax import lax
from jax.experimental import pallas as pl
from jax.experimental.pallas import tpu as pltpu

W = 32
M = 4096
N = 8192
CH = M // W


def _gemm(x, w_mat, scale_x, scale_w):
    m, k = x.shape
    _, n = w_mat.shape
    BM = 512

    def body(sx_ref, sw_ref, x_ref, w_ref, o_ref):
        s = sx_ref[0] * sw_ref[0]
        xb = x_ref[...].astype(jnp.bfloat16)
        wb = w_ref[...].astype(jnp.bfloat16)
        acc = lax.dot_general(
            xb, wb, (((1,), (0,)), ((), ())),
            preferred_element_type=jnp.float32,
        )
        o_ref[...] = acc * s

    return pl.pallas_call(
        body,
        grid=(m // BM,),
        in_specs=[
            pl.BlockSpec(memory_space=pltpu.SMEM),
            pl.BlockSpec(memory_space=pltpu.SMEM),
            pl.BlockSpec((BM, k), lambda i: (i, 0)),
            pl.BlockSpec((k, n), lambda i: (0, 0)),
        ],
        out_specs=pl.BlockSpec((BM, n), lambda i: (i, 0)),
        out_shape=jax.ShapeDtypeStruct((m, n), jnp.float32),
    )(scale_x, scale_w, x, w_mat)


def _allreduce_body(p_ref, o_ref, recv_ref, acc_ref,
                    send_sems, recv_sems, dma_sems, cred_sem):
    my = lax.axis_index("i")
    left = lax.rem(my + W - 1, W)
    right = lax.rem(my + 1, W)

    init = pltpu.make_async_copy(p_ref, o_ref, dma_sems.at[0])
    init.start()
    init.wait()

    for step in range(2 * (W - 1)):
        slot = step % 2
        if step < W - 1:
            c_send = lax.rem(my - step + 2 * W, W)
            c_recv = lax.rem(my - step - 1 + 2 * W, W)
        else:
            t = step - (W - 1)
            c_send = lax.rem(my + 1 - t + 2 * W, W)
            c_recv = lax.rem(my - t + 2 * W, W)

        if step >= 2:
            pl.semaphore_wait(cred_sem, 1)

        rdma = pltpu.make_async_remote_copy(
            src_ref=o_ref.at[pl.ds(c_send * CH, CH)],
            dst_ref=recv_ref.at[slot],
            send_sem=send_sems.at[slot],
            recv_sem=recv_sems.at[slot],
            device_id=(right,),
            device_id_type=pl.DeviceIdType.MESH,
        )
        rdma.start()
        rdma.wait()

        if step < W - 1:
            cp_a = pltpu.make_async_copy(
                o_ref.at[pl.ds(c_recv * CH, CH)], acc_ref, dma_sems.at[0])
            cp_a.start()
            cp_a.wait()
            acc_ref[...] = acc_ref[...] + recv_ref[slot]
            cp_c = pltpu.make_async_copy(
                acc_ref, o_ref.at[pl.ds(c_recv * CH, CH)], dma_sems.at[0])
            cp_c.start()
            cp_c.wait()
        else:
            cp_c = pltpu.make_async_copy(
                recv_ref.at[slot], o_ref.at[pl.ds(c_recv * CH, CH)],
                dma_sems.at[0])
            cp_c.start()
            cp_c.wait()

        pl.semaphore_signal(
            cred_sem, inc=1,
            device_id=(left,), device_id_type=pl.DeviceIdType.MESH)

    pl.semaphore_wait(cred_sem, 2)


def kernel(x, w_mat, scale_x, scale_w):
    partial = _gemm(x, w_mat, scale_x, scale_w)
    out = pl.pallas_call(
        _allreduce_body,
        out_shape=jax.ShapeDtypeStruct((M, N), jnp.float32),
        in_specs=[pl.BlockSpec(memory_space=pl.ANY)],
        out_specs=pl.BlockSpec(memory_space=pl.ANY),
        scratch_shapes=[
            pltpu.VMEM((2, CH, N), jnp.float32),
            pltpu.VMEM((CH, N), jnp.float32),
            pltpu.SemaphoreType.DMA((2,)),
            pltpu.SemaphoreType.DMA((2,)),
            pltpu.SemaphoreType.DMA((2,)),
            pltpu.SemaphoreType.REGULAR,
        ],
    )(partial)
    return out
